# baseline (speedup 1.0000x reference)
import functools

import jax
import jax.numpy as jnp
from jax.experimental import pallas as pl
from jax.experimental.pallas import tpu as pltpu

_WEIGHT_POINT = 100.0
_WEIGHT_EDGE = 100.0


def _edge_loss(e_ref, y_ref, n_ref):
    e = e_ref[...]
    y = y_ref[...]
    n = n_ref[...]
    ii = jax.lax.broadcasted_iota(jnp.int32, e.shape, 0)
    jj = jax.lax.broadcasted_iota(jnp.int32, e.shape, 1)
    nb = n[None, None]
    valid = (ii < nb) & (jj < nb)
    tgt = jj == y[:, None]
    log_p = jnp.maximum(jnp.log(e), -100.0)
    log_1mp = jnp.maximum(jnp.log(1.0 - e), -100.0)
    bce = -jnp.where(tgt, log_p, log_1mp)
    return jnp.sum(jnp.where(valid, bce, 0.0), dtype=jnp.float32)


def _outer(p_hbm, t_hbm, m_hbm, e_ref, y_ref, n_ref, o_ref, cnt_ref,
           *, cp, ce, nsteps, rows, batch):
    o_ref[0, 0] = 0.0
    cnt_ref[0] = 0

    def inner(p_blk, t_blk, m_blk):
        d = (p_blk[...] - t_blk[...]) * m_blk[...]
        o_ref[0, 0] += cp * jnp.sum(d * d, dtype=jnp.float32)

        @pl.when(cnt_ref[0] == 0)
        def _edge():
            o_ref[0, 0] += ce * _edge_loss(e_ref, y_ref, n_ref)

        cnt_ref[0] += 1

    spec = pl.BlockSpec((rows, batch), lambda i: (i, 0))
    pltpu.emit_pipeline(
        inner,
        grid=(nsteps,),
        in_specs=[spec, spec, spec],
    )(p_hbm, t_hbm, m_hbm)


def kernel(points_pred, targets, mask, edges_pred, match_targets, npoints):
    B, C, H, W = points_pred.shape
    F = C * H * W
    M = match_targets.shape[1]
    S, L = 8, B // 8

    pt = jnp.transpose(points_pred, (1, 2, 3, 0)).reshape(F, B)
    tt = jnp.transpose(targets, (1, 2, 3, 0)).reshape(F, B)
    mt = jnp.transpose(mask, (1, 2, 3, 0)).reshape(F, B)
    e4 = jnp.transpose(edges_pred, (2, 1, 0)).reshape(M, M, S, L)
    y3 = jnp.transpose(match_targets, (1, 2, 0)).reshape(M, S, L)
    n2 = npoints.reshape(S, L)

    pt = pltpu.with_memory_space_constraint(pt, pltpu.MemorySpace.HBM)
    tt = pltpu.with_memory_space_constraint(tt, pltpu.MemorySpace.HBM)
    mt = pltpu.with_memory_space_constraint(mt, pltpu.MemorySpace.HBM)

    nsteps = 16
    rows = F // nsteps

    cp = _WEIGHT_POINT / (B * F)
    ce = _WEIGHT_EDGE / (B * M * M)
    body = functools.partial(_outer, cp=cp, ce=ce, nsteps=nsteps, rows=rows,
                             batch=B)

    out = pl.pallas_call(
        body,
        in_specs=[
            pl.BlockSpec(memory_space=pltpu.MemorySpace.HBM),
            pl.BlockSpec(memory_space=pltpu.MemorySpace.HBM),
            pl.BlockSpec(memory_space=pltpu.MemorySpace.HBM),
            pl.BlockSpec(memory_space=pltpu.MemorySpace.VMEM),
            pl.BlockSpec(memory_space=pltpu.MemorySpace.VMEM),
            pl.BlockSpec(memory_space=pltpu.MemorySpace.VMEM),
        ],
        out_specs=pl.BlockSpec(memory_space=pltpu.MemorySpace.SMEM),
        out_shape=jax.ShapeDtypeStruct((1, 1), jnp.float32),
        scratch_shapes=[pltpu.SMEM((1,), jnp.int32)],
    )(pt, tt, mt, e4, y3, n2)
    return out.reshape(())


# speedup vs baseline: 1.8070x; 1.8070x over previous
import functools

import jax
import jax.numpy as jnp
from jax.experimental import pallas as pl
from jax.experimental.pallas import tpu as pltpu

_WEIGHT_POINT = 100.0
_WEIGHT_EDGE = 100.0

_NCHUNKS = 16


def _edge_loss(e_ref, y_ref, n_ref):
    e = e_ref[...]
    y = y_ref[...]
    n = n_ref[...]
    ii = jax.lax.broadcasted_iota(jnp.int32, e.shape, 0)
    jj = jax.lax.broadcasted_iota(jnp.int32, e.shape, 1)
    nb = n[None, None]
    valid = (ii < nb) & (jj < nb)
    tgt = jj == y[:, None]
    log_p = jnp.maximum(jnp.log(e), -100.0)
    log_1mp = jnp.maximum(jnp.log(1.0 - e), -100.0)
    bce = -jnp.where(tgt, log_p, log_1mp)
    return jnp.sum(jnp.where(valid, bce, 0.0), dtype=jnp.float32)


def _outer(p_hbm, t_hbm, m_hbm, e_ref, y_ref, n_ref, o_ref,
           pbuf, tbuf, mbuf, sems, *, cp, ce):
    rows = pbuf.shape[0] // _NCHUNKS

    def copies(k):
        sl = pl.ds(k * rows, rows)
        return (
            pltpu.make_async_copy(p_hbm.at[sl, :], pbuf.at[sl, :],
                                  sems.at[k, 0]),
            pltpu.make_async_copy(t_hbm.at[sl, :], tbuf.at[sl, :],
                                  sems.at[k, 1]),
            pltpu.make_async_copy(m_hbm.at[sl, :], mbuf.at[sl, :],
                                  sems.at[k, 2]),
        )

    for k in range(_NCHUNKS):
        for c in copies(k):
            c.start()

    s_edge = _edge_loss(e_ref, y_ref, n_ref)

    s_point = jnp.float32(0.0)
    for k in range(_NCHUNKS):
        for c in copies(k):
            c.wait()
        sl = pl.ds(k * rows, rows)
        d = (pbuf[sl, :] - tbuf[sl, :]) * mbuf[sl, :]
        s_point += jnp.sum(d * d, dtype=jnp.float32)

    o_ref[0, 0] = cp * s_point + ce * s_edge


def kernel(points_pred, targets, mask, edges_pred, match_targets, npoints):
    B, C, H, W = points_pred.shape
    F = C * H * W
    M = match_targets.shape[1]
    S, L = 8, B // 8

    pt = jnp.transpose(points_pred, (1, 2, 3, 0)).reshape(F, B)
    tt = jnp.transpose(targets, (1, 2, 3, 0)).reshape(F, B)
    mt = jnp.transpose(mask, (1, 2, 3, 0)).reshape(F, B)
    e4 = jnp.transpose(edges_pred, (2, 1, 0)).reshape(M, M, S, L)
    y3 = jnp.transpose(match_targets, (1, 2, 0)).reshape(M, S, L)
    n2 = npoints.reshape(S, L)

    pt = pltpu.with_memory_space_constraint(pt, pltpu.MemorySpace.HBM)
    tt = pltpu.with_memory_space_constraint(tt, pltpu.MemorySpace.HBM)
    mt = pltpu.with_memory_space_constraint(mt, pltpu.MemorySpace.HBM)

    cp = _WEIGHT_POINT / (B * F)
    ce = _WEIGHT_EDGE / (B * M * M)
    body = functools.partial(_outer, cp=cp, ce=ce)

    out = pl.pallas_call(
        body,
        in_specs=[
            pl.BlockSpec(memory_space=pltpu.MemorySpace.HBM),
            pl.BlockSpec(memory_space=pltpu.MemorySpace.HBM),
            pl.BlockSpec(memory_space=pltpu.MemorySpace.HBM),
            pl.BlockSpec(memory_space=pltpu.MemorySpace.VMEM),
            pl.BlockSpec(memory_space=pltpu.MemorySpace.VMEM),
            pl.BlockSpec(memory_space=pltpu.MemorySpace.VMEM),
        ],
        out_specs=pl.BlockSpec(memory_space=pltpu.MemorySpace.SMEM),
        out_shape=jax.ShapeDtypeStruct((1, 1), jnp.float32),
        scratch_shapes=[
            pltpu.VMEM((F, B), jnp.float32),
            pltpu.VMEM((F, B), jnp.float32),
            pltpu.VMEM((F, B), jnp.float32),
            pltpu.SemaphoreType.DMA((_NCHUNKS, 3)),
        ],
    )(pt, tt, mt, e4, y3, n2)
    return out.reshape(())


# speedup vs baseline: 1.8366x; 1.0164x over previous
import functools

import jax
import jax.numpy as jnp
from jax.experimental import pallas as pl
from jax.experimental.pallas import tpu as pltpu

_WEIGHT_POINT = 100.0
_WEIGHT_EDGE = 100.0

_NCHUNKS = 8


def _edge_loss(e_ref, y_ref, n_ref):
    e = e_ref[...]
    y = y_ref[...]
    n = n_ref[...]
    ii = jax.lax.broadcasted_iota(jnp.int32, e.shape, 0)
    jj = jax.lax.broadcasted_iota(jnp.int32, e.shape, 1)
    nb = n[None, None]
    valid = (ii < nb) & (jj < nb)
    tgt = jj == y[:, None]
    log_p = jnp.maximum(jnp.log(e), -100.0)
    log_1mp = jnp.maximum(jnp.log(1.0 - e), -100.0)
    bce = -jnp.where(tgt, log_p, log_1mp)
    return jnp.sum(jnp.where(valid, bce, 0.0), dtype=jnp.float32)


def _outer(p_hbm, t_hbm, m_hbm, e_ref, y_ref, n_ref, o_ref,
           pbuf, tbuf, mbuf, sems, *, cp, ce):
    rows = pbuf.shape[0] // _NCHUNKS

    def copies(k):
        sl = pl.ds(k * rows, rows)
        return (
            pltpu.make_async_copy(p_hbm.at[sl, :], pbuf.at[sl, :],
                                  sems.at[k, 0]),
            pltpu.make_async_copy(t_hbm.at[sl, :], tbuf.at[sl, :],
                                  sems.at[k, 1]),
            pltpu.make_async_copy(m_hbm.at[sl, :], mbuf.at[sl, :],
                                  sems.at[k, 2]),
        )

    for k in range(_NCHUNKS):
        for c in copies(k):
            c.start()

    s_edge = _edge_loss(e_ref, y_ref, n_ref)

    batch = pbuf.shape[1]
    acc = jnp.zeros((8, batch), jnp.float32)
    for k in range(_NCHUNKS):
        for c in copies(k):
            c.wait()
        sl = pl.ds(k * rows, rows)
        d = (pbuf[sl, :] - tbuf[sl, :]) * mbuf[sl, :]
        d2 = (d * d).reshape(rows // 8, 8, batch)
        acc = acc + jnp.sum(d2, axis=0, dtype=jnp.float32)
    s_point = jnp.sum(acc, dtype=jnp.float32)

    o_ref[0, 0] = cp * s_point + ce * s_edge


def kernel(points_pred, targets, mask, edges_pred, match_targets, npoints):
    B, C, H, W = points_pred.shape
    F = C * H * W
    M = match_targets.shape[1]
    S, L = 8, B // 8

    pt = jnp.transpose(points_pred, (1, 2, 3, 0)).reshape(F, B)
    tt = jnp.transpose(targets, (1, 2, 3, 0)).reshape(F, B)
    mt = jnp.transpose(mask, (1, 2, 3, 0)).reshape(F, B)
    e4 = jnp.transpose(edges_pred, (2, 1, 0)).reshape(M, M, S, L)
    y3 = jnp.transpose(match_targets, (1, 2, 0)).reshape(M, S, L)
    n2 = npoints.reshape(S, L)

    pt = pltpu.with_memory_space_constraint(pt, pltpu.MemorySpace.HBM)
    tt = pltpu.with_memory_space_constraint(tt, pltpu.MemorySpace.HBM)
    mt = pltpu.with_memory_space_constraint(mt, pltpu.MemorySpace.HBM)

    cp = _WEIGHT_POINT / (B * F)
    ce = _WEIGHT_EDGE / (B * M * M)
    body = functools.partial(_outer, cp=cp, ce=ce)

    out = pl.pallas_call(
        body,
        in_specs=[
            pl.BlockSpec(memory_space=pltpu.MemorySpace.HBM),
            pl.BlockSpec(memory_space=pltpu.MemorySpace.HBM),
            pl.BlockSpec(memory_space=pltpu.MemorySpace.HBM),
            pl.BlockSpec(memory_space=pltpu.MemorySpace.VMEM),
            pl.BlockSpec(memory_space=pltpu.MemorySpace.VMEM),
            pl.BlockSpec(memory_space=pltpu.MemorySpace.VMEM),
        ],
        out_specs=pl.BlockSpec(memory_space=pltpu.MemorySpace.SMEM),
        out_shape=jax.ShapeDtypeStruct((1, 1), jnp.float32),
        scratch_shapes=[
            pltpu.VMEM((F, B), jnp.float32),
            pltpu.VMEM((F, B), jnp.float32),
            pltpu.VMEM((F, B), jnp.float32),
            pltpu.SemaphoreType.DMA((_NCHUNKS, 3)),
        ],
    )(pt, tt, mt, e4, y3, n2)
    return out.reshape(())
